# Initial kernel scaffold; baseline (speedup 1.0000x reference)
import jax
import jax.numpy as jnp
from jax.experimental import pallas as pl


def kernel(x, weights):
    raise NotImplementedError("write your pallas kernel here")


# speedup vs baseline: 1.9589x; 1.9589x over previous
import jax
import jax.numpy as jnp
from jax.experimental import pallas as pl


def _body(n, x_ref, w_ref, out_ref):
    q = x_ref[:, :n]
    y = x_ref[:, n:]
    z = q * y * w_ref[:]
    m = jnp.max(z, axis=1, keepdims=True)
    m = jnp.maximum(m, 1.0)
    e = jnp.exp(z - m)
    e0 = jnp.exp(1.0 - m)
    s = jnp.sum(e, axis=1, keepdims=True) + e0
    r = 1.0 / s
    out_ref[:, :] = jnp.concatenate([e0 * r, e * r], axis=1)


def kernel(x, weights):
    b, two_n = x.shape
    n = two_n // 2
    rb = 8
    w2d = weights.reshape(1, n)

    import functools
    return pl.pallas_call(
        functools.partial(_body, n),
        grid=(b // rb,),
        in_specs=[
            pl.BlockSpec((rb, two_n), lambda i: (i, 0)),
            pl.BlockSpec((1, n), lambda i: (0, 0)),
        ],
        out_specs=pl.BlockSpec((rb, n + 1), lambda i: (i, 0)),
        out_shape=jax.ShapeDtypeStruct((b, n + 1), jnp.float32),
    )(x, w2d)


# speedup vs baseline: 2.3207x; 1.1846x over previous
import jax
import jax.numpy as jnp
from jax.experimental import pallas as pl


def _body(n, x_ref, w_ref, out_ref):
    q = x_ref[:, :n]
    y = x_ref[:, n:]
    z = q * y * w_ref[:]
    m = jnp.max(z, axis=1, keepdims=True)
    m = jnp.maximum(m, 1.0)
    e = jnp.exp(z - m)
    e0 = jnp.exp(1.0 - m)
    s = jnp.sum(e, axis=1, keepdims=True) + e0
    r = 1.0 / s
    out_ref[:, :] = jnp.concatenate([e0 * r, e * r], axis=1)


def kernel(x, weights):
    b, two_n = x.shape
    n = two_n // 2
    rb = 16
    w2d = weights.reshape(1, n)

    import functools
    return pl.pallas_call(
        functools.partial(_body, n),
        grid=(b // rb,),
        in_specs=[
            pl.BlockSpec((rb, two_n), lambda i: (i, 0)),
            pl.BlockSpec((1, n), lambda i: (0, 0)),
        ],
        out_specs=pl.BlockSpec((rb, n + 1), lambda i: (i, 0)),
        out_shape=jax.ShapeDtypeStruct((b, n + 1), jnp.float32),
    )(x, w2d)


# speedup vs baseline: 2.3944x; 1.0318x over previous
import jax
import jax.numpy as jnp
from jax.experimental import pallas as pl


def _body(n, x_ref, w_ref, out_ref):
    q = x_ref[:, :n]
    y = x_ref[:, n:]
    z = q * y * w_ref[:]
    m = jnp.max(z, axis=1, keepdims=True)
    m = jnp.maximum(m, 1.0)
    e = jnp.exp(z - m)
    e0 = jnp.exp(1.0 - m)
    s = jnp.sum(e, axis=1, keepdims=True) + e0
    r = 1.0 / s
    out_ref[:, :] = jnp.concatenate([e0 * r, e * r], axis=1)


def kernel(x, weights):
    b, two_n = x.shape
    n = two_n // 2
    rb = 32
    w2d = weights.reshape(1, n)

    import functools
    return pl.pallas_call(
        functools.partial(_body, n),
        grid=(b // rb,),
        in_specs=[
            pl.BlockSpec((rb, two_n), lambda i: (i, 0)),
            pl.BlockSpec((1, n), lambda i: (0, 0)),
        ],
        out_specs=pl.BlockSpec((rb, n + 1), lambda i: (i, 0)),
        out_shape=jax.ShapeDtypeStruct((b, n + 1), jnp.float32),
    )(x, w2d)


# speedup vs baseline: 2.3968x; 1.0010x over previous
import functools
import jax
import jax.numpy as jnp
from jax.experimental import pallas as pl
from jax.experimental.pallas import tpu as pltpu


def _body(n, rb, x_ref, w_ref, out_hbm, ob, sem):
    i = pl.program_id(0)
    nsteps = pl.num_programs(0)
    slot = jax.lax.rem(i, 2)

    @pl.when(i >= 2)
    def _wait_prev():
        pltpu.make_async_copy(
            ob.at[slot], out_hbm.at[pl.ds((i - 2) * rb, rb), :], sem.at[slot]
        ).wait()

    q = x_ref[:, :n]
    y = x_ref[:, n:]
    z = q * y * w_ref[:]
    m = jnp.maximum(jnp.max(z, axis=1, keepdims=True), 1.0)
    e = jnp.exp(z - m)
    e0 = jnp.exp(1.0 - m)
    s = jnp.sum(e, axis=1, keepdims=True) + e0
    r = 1.0 / s
    ob[slot] = jnp.concatenate([e0 * r, e * r], axis=1)
    pltpu.make_async_copy(
        ob.at[slot], out_hbm.at[pl.ds(i * rb, rb), :], sem.at[slot]
    ).start()

    @pl.when(i == nsteps - 1)
    def _drain():
        prev = jax.lax.rem(i - 1, 2)
        pltpu.make_async_copy(
            ob.at[prev], out_hbm.at[pl.ds((i - 1) * rb, rb), :], sem.at[prev]
        ).wait()
        pltpu.make_async_copy(
            ob.at[slot], out_hbm.at[pl.ds(i * rb, rb), :], sem.at[slot]
        ).wait()


def kernel(x, weights):
    b, two_n = x.shape
    n = two_n // 2
    rb = 32
    w2d = weights.reshape(1, n)

    return pl.pallas_call(
        functools.partial(_body, n, rb),
        grid=(b // rb,),
        in_specs=[
            pl.BlockSpec((rb, two_n), lambda i: (i, 0)),
            pl.BlockSpec((1, n), lambda i: (0, 0)),
        ],
        out_specs=pl.BlockSpec(memory_space=pl.ANY),
        out_shape=jax.ShapeDtypeStruct((b, n + 1), jnp.float32),
        scratch_shapes=[
            pltpu.VMEM((2, rb, n + 1), jnp.float32),
            pltpu.SemaphoreType.DMA((2,)),
        ],
    )(x, w2d)
